# baseline (speedup 1.0000x reference)
import jax
import jax.numpy as jnp
from jax import lax
from jax.experimental import pallas as pl
from jax.experimental.pallas import tpu as pltpu
from jax.experimental.pallas import tpu_sc as plsc

B = 1024
L = 50
D = 128
E = 8
NW = 32
RPW = B // NW
NP = RPW // 2
NB = 4

_info = plsc.get_sparse_core_info()
_NC, _NS = _info.num_cores, _info.num_subcores


def _body(idx_hbm, loc_hbm, fixvec_hbm, table_hbm, out_hbm,
          idx_v, loc_v, fix_v, b0, b1, b2, b3,
          g0, g1, g2, g3, w0, w1, w2, w3):
    bufs = (b0, b1, b2, b3)
    gs = (g0, g1, g2, g3)
    ws = (w0, w1, w2, w3)
    wid = lax.axis_index("s") * _NC + lax.axis_index("c")
    obase = wid * RPW
    pltpu.sync_copy(idx_hbm.at[wid], idx_v)
    pltpu.sync_copy(loc_hbm.at[wid], loc_v)
    pltpu.sync_copy(fixvec_hbm, fix_v)
    cols = [lax.broadcasted_iota(jnp.int32, (16,), 0) + jnp.int32(16 * c)
            for c in range(D // 16)]

    def gather(p):
        return pltpu.async_copy(table_hbm.at[idx_v.at[jnp.int32(p)]],
                                bufs[p % NB], gs[p % NB])

    def patch(r, slot):
        lvec = loc_v[jnp.int32(r), :]
        for j in range(E):
            rows = lax.gather(
                lvec, jnp.full((16, 1), j, dtype=jnp.int32),
                lax.GatherDimensionNumbers(
                    offset_dims=(), collapsed_slice_dims=(0,),
                    start_index_map=(0,)),
                (1,), mode=lax.GatherScatterMode.PROMISE_IN_BOUNDS)
            for c in range(D // 16):
                val = fix_v[jnp.int32(j), pl.ds(16 * c, 16)]
                plsc.store_scatter(bufs[slot], [rows, cols[c]], val)

    cps = {p: gather(p) for p in range(NB - 1)}
    writes = {}
    for p in range(NP):
        slot = p % NB
        cps[p].wait()
        patch(2 * p, slot)
        patch(2 * p + 1, slot)
        writes[2 * p] = pltpu.async_copy(
            bufs[slot].at[pl.ds(0, L)],
            out_hbm.at[:, obase + jnp.int32(2 * p)], ws[slot])
        writes[2 * p + 1] = pltpu.async_copy(
            bufs[slot].at[pl.ds(L, L)],
            out_hbm.at[:, obase + jnp.int32(2 * p + 1)], ws[slot])
        nxt = p + NB - 1
        if nxt < NP:
            if p >= 1:
                writes[2 * (p - 1)].wait()
                writes[2 * (p - 1) + 1].wait()
            cps[nxt] = gather(nxt)
    for p in range(NP - NB, NP):
        writes[2 * p].wait()
        writes[2 * p + 1].wait()


def kernel(input_ids, fix_offsets, table, fix_vec):
    idx = input_ids.astype(jnp.int32).reshape(NW, NP, 2 * L)
    start = fix_offsets.astype(jnp.int32) + 1
    loc = (start[:, None] + jnp.arange(16, dtype=jnp.int32)[None, :]
           + (jnp.arange(B, dtype=jnp.int32) % 2 * L)[:, None]
           ).reshape(NW, RPW, 16)
    mesh = plsc.VectorSubcoreMesh(core_axis_name="c", subcore_axis_name="s")
    run = pl.kernel(
        _body,
        mesh=mesh,
        out_type=jax.ShapeDtypeStruct((L, B, D), jnp.float32),
        scratch_types=(
            [pltpu.VMEM((NP, 2 * L), jnp.int32),
             pltpu.VMEM((RPW, 16), jnp.int32),
             pltpu.VMEM((E, D), jnp.float32)]
            + [pltpu.VMEM((2 * L, D), jnp.float32)] * NB
            + [pltpu.SemaphoreType.DMA] * (2 * NB)
        ),
        compiler_params=pltpu.CompilerParams(
            needs_layout_passes=False, use_tc_tiling_on_sc=False),
    )
    out = run(idx, loc, fix_vec, table)
    return out.transpose(1, 0, 2)


# speedup vs baseline: 1.0987x; 1.0987x over previous
import jax
import jax.numpy as jnp
from jax import lax
from jax.experimental import pallas as pl
from jax.experimental.pallas import tpu as pltpu
from jax.experimental.pallas import tpu_sc as plsc

B = 1024
L = 50
D = 128
E = 8
NW = 32
RPW = B // NW
NP = RPW // 2
NB = 4

_info = plsc.get_sparse_core_info()
_NC, _NS = _info.num_cores, _info.num_subcores


def _body(idx_hbm, loc_hbm, fixvec_hbm, table_hbm, out_hbm,
          idx_v, loc_v, fix_v, b0, b1, b2, b3,
          g0, g1, g2, g3, w0, w1, w2, w3):
    bufs = (b0, b1, b2, b3)
    gs = (g0, g1, g2, g3)
    ws = (w0, w1, w2, w3)
    wid = lax.axis_index("s") * _NC + lax.axis_index("c")
    obase = wid * RPW
    pltpu.sync_copy(idx_hbm.at[wid], idx_v)
    pltpu.sync_copy(loc_hbm.at[wid], loc_v)
    pltpu.sync_copy(fixvec_hbm, fix_v)
    cols = [lax.broadcasted_iota(jnp.int32, (16,), 0) + jnp.int32(16 * c)
            for c in range(D // 16)]

    def gather(p, slot):
        return pltpu.async_copy(table_hbm.at[idx_v.at[p]], bufs[slot],
                                gs[slot])

    def patch(r, slot):
        lvec = loc_v[r, :]
        for j in range(E):
            rows = lax.gather(
                lvec, jnp.full((16, 1), j, dtype=jnp.int32),
                lax.GatherDimensionNumbers(
                    offset_dims=(), collapsed_slice_dims=(0,),
                    start_index_map=(0,)),
                (1,), mode=lax.GatherScatterMode.PROMISE_IN_BOUNDS)
            for c in range(D // 16):
                val = fix_v[jnp.int32(j), pl.ds(16 * c, 16)]
                plsc.store_scatter(bufs[slot], [rows, cols[c]], val)

    def step(p, slot, wait_prev_write):
        pltpu.make_async_copy(table_hbm.at[idx_v.at[p]], bufs[slot],
                              gs[slot]).wait()
        patch(jnp.int32(2) * p, slot)
        patch(jnp.int32(2) * p + jnp.int32(1), slot)
        pltpu.async_copy(bufs[slot].at[pl.ds(0, L)],
                         out_hbm.at[:, obase + jnp.int32(2) * p], ws[slot])
        pltpu.async_copy(bufs[slot].at[pl.ds(L, L)],
                         out_hbm.at[:, obase + jnp.int32(2) * p
                                    + jnp.int32(1)], ws[slot])
        nslot = (slot + NB - 1) % NB
        if wait_prev_write:
            for _ in range(2):
                pltpu.make_async_copy(bufs[nslot].at[pl.ds(0, L)],
                                      out_hbm.at[:, obase], ws[nslot]).wait()
        gather(p + NB - 1, nslot)

    for p in range(NB - 1):
        gather(jnp.int32(p), p)
    step(jnp.int32(0), 0, False)

    def outer(i, carry):
        ii = i.astype(jnp.int32)
        for b in range(NB):
            step(jnp.int32(1 + b) + ii * jnp.int32(NB), (1 + b) % NB, True)
        return carry
    lax.fori_loop(jnp.int32(0), jnp.int32((NP - NB) // NB), outer,
                  jnp.int32(0))

    for p in range(NP - NB + 1, NP):
        slot = p % NB
        pltpu.make_async_copy(table_hbm.at[idx_v.at[jnp.int32(p)]],
                              bufs[slot], gs[slot]).wait()
        patch(jnp.int32(2 * p), slot)
        patch(jnp.int32(2 * p + 1), slot)
        pltpu.async_copy(bufs[slot].at[pl.ds(0, L)],
                         out_hbm.at[:, obase + jnp.int32(2 * p)], ws[slot])
        pltpu.async_copy(bufs[slot].at[pl.ds(L, L)],
                         out_hbm.at[:, obase + jnp.int32(2 * p + 1)],
                         ws[slot])
    for p in range(NP - NB, NP):
        slot = p % NB
        for _ in range(2):
            pltpu.make_async_copy(bufs[slot].at[pl.ds(0, L)],
                                  out_hbm.at[:, obase], ws[slot]).wait()


def kernel(input_ids, fix_offsets, table, fix_vec):
    idx = input_ids.astype(jnp.int32).reshape(NW, NP, 2 * L)
    start = fix_offsets.astype(jnp.int32) + 1
    loc = (start[:, None] + jnp.arange(16, dtype=jnp.int32)[None, :]
           + (jnp.arange(B, dtype=jnp.int32) % 2 * L)[:, None]
           ).reshape(NW, RPW, 16)
    mesh = plsc.VectorSubcoreMesh(core_axis_name="c", subcore_axis_name="s")
    run = pl.kernel(
        _body,
        mesh=mesh,
        out_type=jax.ShapeDtypeStruct((L, B, D), jnp.float32),
        scratch_types=(
            [pltpu.VMEM((NP, 2 * L), jnp.int32),
             pltpu.VMEM((RPW, 16), jnp.int32),
             pltpu.VMEM((E, D), jnp.float32)]
            + [pltpu.VMEM((2 * L, D), jnp.float32)] * NB
            + [pltpu.SemaphoreType.DMA] * (2 * NB)
        ),
        compiler_params=pltpu.CompilerParams(
            needs_layout_passes=False, use_tc_tiling_on_sc=False),
    )
    out = run(idx, loc, fix_vec, table)
    return out.transpose(1, 0, 2)
